# Initial kernel scaffold; baseline (speedup 1.0000x reference)
import jax
import jax.numpy as jnp
from jax.experimental import pallas as pl


def kernel(token_ids, embedding_table):
    raise NotImplementedError("write your pallas kernel here")


# speedup vs baseline: 1.8447x; 1.8447x over previous
import functools

import jax
import jax.numpy as jnp
from jax import lax
from jax.experimental import pallas as pl
from jax.experimental.pallas import tpu as pltpu
from jax.experimental.pallas import tpu_sc as plsc

_NUM_CORES = 2
_NUM_SUBCORES = 16
_NUM_WORKERS = _NUM_CORES * _NUM_SUBCORES
_IDX_PER_STREAM = 128


@functools.partial(jax.jit, static_argnames=("chunk", "n_chunks"))
def _sc_embedding_lookup(idx2d, table, *, chunk, n_chunks):
    n_rows_total, _ = idx2d.shape
    n = n_rows_total * _IDX_PER_STREAM
    d = table.shape[1]
    k = chunk // _IDX_PER_STREAM
    per_worker = chunk * n_chunks
    assert per_worker * _NUM_WORKERS == n

    mesh = plsc.VectorSubcoreMesh(core_axis_name="c", subcore_axis_name="s")

    @functools.partial(
        pl.kernel,
        out_type=jax.ShapeDtypeStruct((n, d), jnp.float32),
        mesh=mesh,
        scratch_types=[
            pltpu.VMEM((k, _IDX_PER_STREAM), jnp.int32),
            pltpu.VMEM((chunk, d), jnp.float32),
            pltpu.SemaphoreType.DMA,
        ],
        compiler_params=pltpu.CompilerParams(use_tc_tiling_on_sc=False),
    )
    def lookup(idx_hbm, tab_hbm, out_hbm, idx_v, rows_v, gsem):
        wid = lax.axis_index("s") * _NUM_CORES + lax.axis_index("c")
        row_base = wid * (per_worker // _IDX_PER_STREAM)
        out_base = wid * per_worker

        def body(c, carry):
            pltpu.sync_copy(idx_hbm.at[pl.ds(row_base + c * k, k)], idx_v)
            copies = []
            for j in range(k):
                copies.append(
                    pltpu.async_copy(
                        tab_hbm.at[idx_v.at[j]],
                        rows_v.at[pl.ds(j * _IDX_PER_STREAM, _IDX_PER_STREAM)],
                        gsem,
                    )
                )
            for cp in copies:
                cp.wait()
            pltpu.sync_copy(
                rows_v, out_hbm.at[pl.ds(out_base + c * chunk, chunk)]
            )
            return carry

        lax.fori_loop(0, n_chunks, body, 0)

    return lookup(idx2d, table)


def kernel(token_ids, embedding_table):
    batch, seq = token_ids.shape
    _, d = embedding_table.shape
    n = batch * seq
    ids = token_ids.reshape(n).astype(jnp.int32)
    idx2d = ids.reshape(n // _IDX_PER_STREAM, _IDX_PER_STREAM)
    chunk = 1024
    n_chunks = n // (_NUM_WORKERS * chunk)
    assert chunk * n_chunks * _NUM_WORKERS == n
    out = _sc_embedding_lookup(idx2d, embedding_table, chunk=chunk, n_chunks=n_chunks)
    return out.reshape(batch, seq, d)


# speedup vs baseline: 1.8750x; 1.0164x over previous
import functools

import jax
import jax.numpy as jnp
from jax import lax
from jax.experimental import pallas as pl
from jax.experimental.pallas import tpu as pltpu
from jax.experimental.pallas import tpu_sc as plsc

_NUM_CORES = 2
_NUM_SUBCORES = 16
_NUM_WORKERS = _NUM_CORES * _NUM_SUBCORES
_LANE = 128


@functools.partial(jax.jit, static_argnames=("chunk", "n_chunks"))
def _sc_embedding_lookup(idx2d, table, *, chunk, n_chunks):
    n_rows_total, _ = idx2d.shape
    n = n_rows_total * _LANE
    d = table.shape[1]
    k = chunk // _LANE
    per_worker = chunk * n_chunks
    idx_rows = per_worker // _LANE
    assert per_worker * _NUM_WORKERS == n
    assert n_chunks % 2 == 0

    mesh = plsc.VectorSubcoreMesh(core_axis_name="c", subcore_axis_name="s")

    @functools.partial(
        pl.kernel,
        out_type=jax.ShapeDtypeStruct((n, d), jnp.float32),
        mesh=mesh,
        scratch_types=[
            pltpu.VMEM((idx_rows, _LANE), jnp.int32),
            pltpu.VMEM((2, chunk, d), jnp.float32),
            pltpu.SemaphoreType.DMA,
            pltpu.SemaphoreType.DMA,
            pltpu.SemaphoreType.DMA,
            pltpu.SemaphoreType.DMA,
        ],
        compiler_params=pltpu.CompilerParams(use_tc_tiling_on_sc=False),
    )
    def lookup(idx_hbm, tab_hbm, out_hbm, idx_v, rows_v, g0, g1, o0, o1):
        wid = lax.axis_index("s") * _NUM_CORES + lax.axis_index("c")
        out_base = wid * per_worker
        gsem = (g0, g1)
        osem = (o0, o1)

        def fire_gathers(s, slot):
            for j in range(k):
                pltpu.async_copy(
                    tab_hbm.at[idx_v.at[s * k + j]],
                    rows_v.at[slot, pl.ds(j * _LANE, _LANE)],
                    gsem[slot],
                )

        def drain_gathers(slot):
            pltpu.make_async_copy(
                tab_hbm.at[pl.ds(0, chunk)], rows_v.at[slot], gsem[slot]
            ).wait()

        def fire_out(s, slot):
            pltpu.async_copy(
                rows_v.at[slot],
                out_hbm.at[pl.ds(out_base + s * chunk, chunk)],
                osem[slot],
            )

        def drain_out(slot):
            pltpu.make_async_copy(
                rows_v.at[slot], out_hbm.at[pl.ds(0, chunk)], osem[slot]
            ).wait()

        pltpu.sync_copy(idx_hbm.at[pl.ds(wid * idx_rows, idx_rows)], idx_v)

        fire_gathers(0, 0)
        fire_gathers(1, 1)
        drain_gathers(0)
        fire_out(0, 0)

        def body(i, carry):
            s0 = 2 + 2 * i
            for b in (0, 1):
                s = s0 + b
                drain_out(b)
                fire_gathers(s, b)
                drain_gathers(1 - b)
                fire_out(s - 1, 1 - b)
            return carry

        lax.fori_loop(0, (n_chunks - 2) // 2, body, 0)

        drain_gathers(1)
        fire_out(n_chunks - 1, 1)
        drain_out(0)
        drain_out(1)

    return lookup(idx2d, table)


def kernel(token_ids, embedding_table):
    batch, seq = token_ids.shape
    _, d = embedding_table.shape
    n = batch * seq
    ids = token_ids.reshape(n).astype(jnp.int32)
    idx2d = ids.reshape(n // _LANE, _LANE)
    chunk = 640
    n_chunks = n // (_NUM_WORKERS * chunk)
    assert chunk * n_chunks * _NUM_WORKERS == n
    out = _sc_embedding_lookup(idx2d, embedding_table, chunk=chunk, n_chunks=n_chunks)
    return out.reshape(batch, seq, d)
